# baseline (speedup 1.0000x reference)
import functools

import jax
import jax.numpy as jnp
from jax import lax
from jax.experimental import pallas as pl
from jax.experimental.pallas import tpu as pltpu
from jax.experimental.pallas import tpu_sc as plsc

N_LAYERS = 12
BEAM = 3
TOPK = 3
VOCAB = 100000
HIST = 20
KV_CHUNKS = 8
NEG = -3.4e38


def _beam_body(logits_ref, save_id_ref, rp_ref, prev_ref, pen_ref,
               tbi_ref, nsi_ref, rp_out_ref, tbp_ref, mli_ref, srcrows_ref,
               cand_v, cand_i):
    x = logits_ref[...] * rp_ref[...]
    m = jnp.max(x, axis=1, keepdims=True)
    lse = jnp.log(jnp.sum(jnp.exp(x - m), axis=1, keepdims=True))
    lg = x - m - lse

    iota = lax.broadcasted_iota(jnp.int32, (BEAM, VOCAB), 1)
    cur = lg
    for k in range(TOPK):
        mx = jnp.max(cur, axis=1, keepdims=True)
        am = jnp.min(jnp.where(cur == mx, iota, VOCAB), axis=1,
                     keepdims=True)
        for r in range(BEAM):
            cand_v[r * TOPK + k] = mx[r, 0] + prev_ref[r, 0]
            cand_i[r * TOPK + k] = am[r, 0]
        if k < TOPK - 1:
            cur = jnp.where(iota == am, NEG, cur)

    col_iota = lax.broadcasted_iota(jnp.int32, (1, VOCAB), 1)
    b_sel = []
    for j in range(BEAM):
        bv = cand_v[0]
        bc = jnp.int32(0)
        for c in range(1, BEAM * TOPK):
            take = cand_v[c] > bv
            bv = jnp.where(take, cand_v[c], bv)
            bc = jnp.where(take, jnp.int32(c), bc)
        cand_v[bc] = NEG
        b_j = bc // TOPK
        t_j = cand_i[bc]
        b_sel.append(b_j)
        tbp_ref[j, 0] = bv
        tbi_ref[j, 0] = t_j
        if j == 0:
            mli_ref[0] = t_j
        for t in range(HIST):
            nsi_ref[j, t] = save_id_ref[b_j, t]
        nsi_ref[j, HIST] = t_j
        row = rp_ref[pl.ds(b_j, 1), :]
        row = jnp.where(col_iota == t_j, row * pen_ref[0], row)
        rp_out_ref[pl.ds(j, 1), :] = row

    packed = b_sel[0] + 4 * b_sel[1] + 16 * b_sel[2]
    for j in range(16):
        srcrows_ref[j] = packed


def _gather_body(bidx_ref, *refs):
    del bidx_ref
    n = len(refs) // 2
    for l in range(n):
        refs[n + l][...] = refs[l][...]


def _gather_tc(kvs, beam_index):
    n = len(kvs)
    shape = kvs[0].shape
    flat = shape[1] * shape[2] * shape[3]
    rows = flat // 128
    chunk = rows // KV_CHUNKS
    kvs2 = [kv.reshape(BEAM, rows, 128) for kv in kvs]

    def in_map(b, c, bidx):
        return (bidx[b], c, 0)

    def out_map(b, c, bidx):
        return (b, c, 0)

    block = (1, chunk, 128)
    grid_spec = pltpu.PrefetchScalarGridSpec(
        num_scalar_prefetch=1,
        grid=(BEAM, KV_CHUNKS),
        in_specs=[pl.BlockSpec(block, in_map) for _ in range(n)],
        out_specs=[pl.BlockSpec(block, out_map) for _ in range(n)],
    )
    outs = pl.pallas_call(
        _gather_body,
        grid_spec=grid_spec,
        out_shape=[jax.ShapeDtypeStruct((BEAM, rows, 128), kv.dtype)
                   for kv in kvs2],
    )(beam_index, *kvs2)
    return [o.reshape(shape) for o in outs]


def _gather_sc(kvs, beam_index16):
    n = len(kvs)
    shape = kvs[0].shape
    nh, sl, hd = shape[1], shape[2], shape[3]
    ch = 128
    nchunk = sl // ch
    items = BEAM * nh * nchunk
    nw = 32
    ipt = items // nw
    nb = 7
    T = n * ipt
    mesh = plsc.VectorSubcoreMesh(core_axis_name="c", subcore_axis_name="s")

    @functools.partial(
        pl.kernel,
        out_type=[jax.ShapeDtypeStruct(shape, jnp.float32)
                  for _ in range(n)],
        mesh=mesh,
        compiler_params=pltpu.CompilerParams(needs_layout_passes=False),
        scratch_types=[
            pltpu.VMEM((16,), jnp.int32),
            pltpu.VMEM((nb * ch, hd), jnp.float32),
            pltpu.SemaphoreType.DMA((nb,)),
            pltpu.SemaphoreType.DMA((nb,)),
        ])
    def k(bi_hbm, *refs):
        kv_refs = refs[:n]
        out_refs = refs[n:2 * n]
        bi_v, ring, in_sems, out_sems = refs[2 * n:]
        wid = lax.axis_index("s") * 2 + lax.axis_index("c")
        pltpu.sync_copy(bi_hbm, bi_v)
        packed = jnp.max(bi_v[...])
        b_sc = [packed & 3, (packed >> 2) & 3, (packed >> 4) & 3]

        coords = []
        for i in range(ipt):
            q = wid * ipt + i
            pair = q // nchunk
            cc = q - pair * nchunk
            j = pair // nh
            h = pair - j * nh
            b_src = jnp.where(j == 0, b_sc[0],
                              jnp.where(j == 1, b_sc[1], b_sc[2]))
            coords.append((j, h, cc, b_src))

        gh = [None] * T
        oh = [None] * T

        def start_out(t):
            l, i = divmod(t, ipt)
            j, h, cc, _ = coords[i]
            oh[t] = pltpu.async_copy(
                ring.at[pl.ds((t % nb) * ch, ch)],
                out_refs[l].at[j, h, pl.ds(cc * ch, ch), :],
                out_sems.at[t % nb])

        lag = 3
        for t in range(T):
            l, i = divmod(t, ipt)
            j, h, cc, b_src = coords[i]
            if t >= nb:
                oh[t - nb].wait()
            gh[t] = pltpu.async_copy(
                kv_refs[l].at[b_src, h, pl.ds(cc * ch, ch), :],
                ring.at[pl.ds((t % nb) * ch, ch)],
                in_sems.at[t % nb])
            if t >= lag:
                gh[t - lag].wait()
                start_out(t - lag)
        for t in range(T - lag, T):
            gh[t].wait()
            start_out(t)
        for t in range(T - nb, T):
            oh[t].wait()

    return list(k(beam_index16, *kvs))


@jax.jit
def _run(kvs, logits, save_id, repeat_penality, previous_prob, penality_value):
    small_out_shape = [
        jax.ShapeDtypeStruct((BEAM, 1), jnp.int32),
        jax.ShapeDtypeStruct((BEAM, HIST + 1), jnp.int32),
        jax.ShapeDtypeStruct((BEAM, VOCAB), jnp.float32),
        jax.ShapeDtypeStruct((BEAM, 1), jnp.float32),
        jax.ShapeDtypeStruct((1,), jnp.int32),
        jax.ShapeDtypeStruct((16,), jnp.int32),
    ]
    vmem = pl.BlockSpec(memory_space=pltpu.MemorySpace.VMEM)
    smem = pl.BlockSpec(memory_space=pltpu.SMEM)
    tbi, nsi, rp_out, tbp, mli, src_rows = pl.pallas_call(
        _beam_body,
        out_shape=small_out_shape,
        in_specs=[vmem, smem, vmem, smem, smem],
        out_specs=[smem, smem, vmem, smem, smem, smem],
        scratch_shapes=[
            pltpu.SMEM((BEAM * TOPK,), jnp.float32),
            pltpu.SMEM((BEAM * TOPK,), jnp.int32),
        ],
    )(logits, save_id, repeat_penality, previous_prob, penality_value)
    save_kv = _gather_sc(kvs, src_rows)
    return (*save_kv, tbi, nsi, rp_out, tbp, mli)


def kernel(kv_0, kv_1, kv_2, kv_3, kv_4, kv_5, kv_6, kv_7, kv_8, kv_9,
           kv_10, kv_11, logits, save_id, repeat_penality, previous_prob,
           penality_value, beam_size, topK):
    kvs = (kv_0, kv_1, kv_2, kv_3, kv_4, kv_5, kv_6, kv_7, kv_8, kv_9,
           kv_10, kv_11)
    return _run(kvs, logits, save_id, repeat_penality, previous_prob,
                penality_value)


# speedup vs baseline: 1.0543x; 1.0543x over previous
import functools

import jax
import jax.numpy as jnp
from jax import lax
from jax.experimental import pallas as pl
from jax.experimental.pallas import tpu as pltpu
from jax.experimental.pallas import tpu_sc as plsc

N_LAYERS = 12
BEAM = 3
TOPK = 3
VOCAB = 100000
HIST = 20
SC_LAYERS = 0
NEG = -3.4e38


def _beam_body(logits_ref, save_id_ref, rp_ref, prev_ref, pen_ref,
               tbi_ref, nsi_ref, rp_out_ref, tbp_ref, mli_ref, srcrows_ref,
               cand_v, cand_i):
    x = logits_ref[...] * rp_ref[...]
    m = jnp.max(x, axis=1, keepdims=True)
    lse = jnp.log(jnp.sum(jnp.exp(x - m), axis=1, keepdims=True))
    lg = x - m - lse

    iota = lax.broadcasted_iota(jnp.int32, (BEAM, VOCAB), 1)
    cur = lg
    for k in range(TOPK):
        mx = jnp.max(cur, axis=1, keepdims=True)
        am = jnp.min(jnp.where(cur == mx, iota, VOCAB), axis=1,
                     keepdims=True)
        for r in range(BEAM):
            cand_v[r * TOPK + k] = mx[r, 0] + prev_ref[r, 0]
            cand_i[r * TOPK + k] = am[r, 0]
        if k < TOPK - 1:
            cur = jnp.where(iota == am, NEG, cur)

    col_iota = lax.broadcasted_iota(jnp.int32, (1, VOCAB), 1)
    b_sel = []
    for j in range(BEAM):
        bv = cand_v[0]
        bc = jnp.int32(0)
        for c in range(1, BEAM * TOPK):
            take = cand_v[c] > bv
            bv = jnp.where(take, cand_v[c], bv)
            bc = jnp.where(take, jnp.int32(c), bc)
        cand_v[bc] = NEG
        b_j = bc // TOPK
        t_j = cand_i[bc]
        b_sel.append(b_j)
        tbp_ref[j, 0] = bv
        tbi_ref[j, 0] = t_j
        if j == 0:
            mli_ref[0] = t_j
        for t in range(HIST):
            nsi_ref[j, t] = save_id_ref[b_j, t]
        nsi_ref[j, HIST] = t_j
        row = rp_ref[pl.ds(b_j, 1), :]
        row = jnp.where(col_iota == t_j, row * pen_ref[0], row)
        rp_out_ref[pl.ds(j, 1), :] = row

    packed = b_sel[0] + 4 * b_sel[1] + 16 * b_sel[2]
    for j in range(BEAM):
        srcrows_ref[j] = b_sel[j]
    for j in range(BEAM, 16):
        srcrows_ref[j] = packed


def _gather_tc(kvs, beam_index16):
    n = len(kvs)
    shape = kvs[0].shape
    nh = shape[1]
    units = n * BEAM * nh
    nb = 24
    lag = 12

    def body(bi_ref, *refs):
        kv_refs = refs[:n]
        out_refs = refs[n:2 * n]
        ring, in_sems, out_sems = refs[2 * n:]
        b = [bi_ref[j] for j in range(BEAM)]
        gh = [None] * units
        oh = [None] * units

        def start_out(u):
            l, r = divmod(u, BEAM * nh)
            j, h = divmod(r, nh)
            cp = pltpu.make_async_copy(ring.at[u % nb],
                                       out_refs[l].at[j, h],
                                       out_sems.at[u % nb])
            cp.start()
            oh[u] = cp

        for u in range(units):
            l, r = divmod(u, BEAM * nh)
            j, h = divmod(r, nh)
            if u >= nb:
                oh[u - nb].wait()
            cp = pltpu.make_async_copy(kv_refs[l].at[b[j], h],
                                       ring.at[u % nb],
                                       in_sems.at[u % nb])
            cp.start()
            gh[u] = cp
            if u >= lag:
                gh[u - lag].wait()
                start_out(u - lag)
        for u in range(units - lag, units):
            gh[u].wait()
            start_out(u)
        for u in range(units - nb, units):
            oh[u].wait()

    hbm = pl.BlockSpec(memory_space=pl.ANY)
    smem = pl.BlockSpec(memory_space=pltpu.SMEM)
    return list(pl.pallas_call(
        body,
        out_shape=[jax.ShapeDtypeStruct(shape, jnp.float32)
                   for _ in range(n)],
        in_specs=[smem] + [hbm] * n,
        out_specs=[hbm] * n,
        scratch_shapes=[
            pltpu.VMEM((nb, shape[2], shape[3]), jnp.float32),
            pltpu.SemaphoreType.DMA((nb,)),
            pltpu.SemaphoreType.DMA((nb,)),
        ],
    )(beam_index16, *kvs))


def _gather_sc(kvs, beam_index16):
    n = len(kvs)
    shape = kvs[0].shape
    nh, sl, hd = shape[1], shape[2], shape[3]
    ch = 128
    nchunk = sl // ch
    items = BEAM * nh * nchunk
    nw = 32
    ipt = items // nw
    nb = 7
    T = n * ipt
    mesh = plsc.VectorSubcoreMesh(core_axis_name="c", subcore_axis_name="s")

    @functools.partial(
        pl.kernel,
        out_type=[jax.ShapeDtypeStruct(shape, jnp.float32)
                  for _ in range(n)],
        mesh=mesh,
        compiler_params=pltpu.CompilerParams(needs_layout_passes=False),
        scratch_types=[
            pltpu.VMEM((16,), jnp.int32),
            pltpu.VMEM((nb * ch, hd), jnp.float32),
            pltpu.SemaphoreType.DMA((nb,)),
            pltpu.SemaphoreType.DMA((nb,)),
        ])
    def k(bi_hbm, *refs):
        kv_refs = refs[:n]
        out_refs = refs[n:2 * n]
        bi_v, ring, in_sems, out_sems = refs[2 * n:]
        wid = lax.axis_index("s") * 2 + lax.axis_index("c")
        pltpu.sync_copy(bi_hbm, bi_v)
        packed = jnp.max(bi_v[...])
        b_sc = [packed & 3, (packed >> 2) & 3, (packed >> 4) & 3]

        coords = []
        for i in range(ipt):
            q = wid * ipt + i
            pair = q // nchunk
            cc = q - pair * nchunk
            j = pair // nh
            h = pair - j * nh
            b_src = jnp.where(j == 0, b_sc[0],
                              jnp.where(j == 1, b_sc[1], b_sc[2]))
            coords.append((j, h, cc, b_src))

        gh = [None] * T
        oh = [None] * T

        def start_out(t):
            l, i = divmod(t, ipt)
            j, h, cc, _ = coords[i]
            oh[t] = pltpu.async_copy(
                ring.at[pl.ds((t % nb) * ch, ch)],
                out_refs[l].at[j, h, pl.ds(cc * ch, ch), :],
                out_sems.at[t % nb])

        lag = 3
        for t in range(T):
            l, i = divmod(t, ipt)
            j, h, cc, b_src = coords[i]
            if t >= nb:
                oh[t - nb].wait()
            gh[t] = pltpu.async_copy(
                kv_refs[l].at[b_src, h, pl.ds(cc * ch, ch), :],
                ring.at[pl.ds((t % nb) * ch, ch)],
                in_sems.at[t % nb])
            if t >= lag:
                gh[t - lag].wait()
                start_out(t - lag)
        for t in range(T - lag, T):
            gh[t].wait()
            start_out(t)
        for t in range(T - nb, T):
            oh[t].wait()

    return list(k(beam_index16, *kvs))


@jax.jit
def _run(kvs, logits, save_id, repeat_penality, previous_prob, penality_value):
    small_out_shape = [
        jax.ShapeDtypeStruct((BEAM, 1), jnp.int32),
        jax.ShapeDtypeStruct((BEAM, HIST + 1), jnp.int32),
        jax.ShapeDtypeStruct((BEAM, VOCAB), jnp.float32),
        jax.ShapeDtypeStruct((BEAM, 1), jnp.float32),
        jax.ShapeDtypeStruct((1,), jnp.int32),
        jax.ShapeDtypeStruct((16,), jnp.int32),
    ]
    vmem = pl.BlockSpec(memory_space=pltpu.MemorySpace.VMEM)
    smem = pl.BlockSpec(memory_space=pltpu.SMEM)
    tbi, nsi, rp_out, tbp, mli, src_rows = pl.pallas_call(
        _beam_body,
        out_shape=small_out_shape,
        in_specs=[vmem, smem, vmem, smem, smem],
        out_specs=[smem, smem, vmem, smem, smem, smem],
        scratch_shapes=[
            pltpu.SMEM((BEAM * TOPK,), jnp.float32),
            pltpu.SMEM((BEAM * TOPK,), jnp.int32),
        ],
    )(logits, save_id, repeat_penality, previous_prob, penality_value)
    if SC_LAYERS == 0:
        save_kv = _gather_tc(kvs, src_rows)
    elif SC_LAYERS == N_LAYERS:
        save_kv = _gather_sc(kvs, src_rows)
    else:
        sc_part = _gather_sc(kvs[:SC_LAYERS], src_rows)
        tc_part = _gather_tc(kvs[SC_LAYERS:], src_rows)
        save_kv = list(sc_part) + list(tc_part)
    return (*save_kv, tbi, nsi, rp_out, tbp, mli)


def kernel(kv_0, kv_1, kv_2, kv_3, kv_4, kv_5, kv_6, kv_7, kv_8, kv_9,
           kv_10, kv_11, logits, save_id, repeat_penality, previous_prob,
           penality_value, beam_size, topK):
    kvs = (kv_0, kv_1, kv_2, kv_3, kv_4, kv_5, kv_6, kv_7, kv_8, kv_9,
           kv_10, kv_11)
    return _run(kvs, logits, save_id, repeat_penality, previous_prob,
                penality_value)


# speedup vs baseline: 1.1766x; 1.1160x over previous
import jax
import jax.numpy as jnp
from jax import lax
from jax.experimental import pallas as pl
from jax.experimental.pallas import tpu as pltpu

N_LAYERS = 12
BEAM = 3
TOPK = 3
VOCAB = 100000
HIST = 20
NEG = -3.4e38
NB = 27
LAG = 12


def _body(logits_ref, save_id_ref, rp_ref, prev_ref, pen_ref, *refs):
    kv_refs = refs[:N_LAYERS]
    out_refs = refs[N_LAYERS:2 * N_LAYERS]
    (tbi_ref, nsi_ref, rp_out_ref, tbp_ref, mli_ref,
     cand_v, cand_i, ring, in_sems, out_sems) = refs[2 * N_LAYERS:]

    x = logits_ref[...] * rp_ref[...]
    m = jnp.max(x, axis=1, keepdims=True)
    lse = jnp.log(jnp.sum(jnp.exp(x - m), axis=1, keepdims=True))
    lg = x - m - lse

    iota = lax.broadcasted_iota(jnp.int32, (BEAM, VOCAB), 1)
    cur = lg
    for k in range(TOPK):
        mx = jnp.max(cur, axis=1, keepdims=True)
        am = jnp.min(jnp.where(cur == mx, iota, VOCAB), axis=1,
                     keepdims=True)
        mxp = mx + prev_ref[...]
        for r in range(BEAM):
            cand_v[r * TOPK + k] = mxp[r, 0]
            cand_i[r * TOPK + k] = am[r, 0]
        if k < TOPK - 1:
            cur = jnp.where(iota == am, NEG, cur)

    b_sel = []
    t_sel = []
    v_sel = []
    for j in range(BEAM):
        bv = cand_v[0]
        bc = jnp.int32(0)
        for c in range(1, BEAM * TOPK):
            take = cand_v[c] > bv
            bv = jnp.where(take, cand_v[c], bv)
            bc = jnp.where(take, jnp.int32(c), bc)
        cand_v[bc] = NEG
        b_sel.append(bc // TOPK)
        t_sel.append(cand_i[bc])
        v_sel.append(bv)
    mli_ref[0] = t_sel[0]

    b = b_sel
    nh = 12
    units = N_LAYERS * BEAM * nh
    f = [jnp.int32(0),
         jnp.where(b[1] == b[0], jnp.int32(0), jnp.int32(1)),
         jnp.where(b[2] == b[0], jnp.int32(0),
                   jnp.where(b[2] == b[1], jnp.int32(1), jnp.int32(2)))]
    gh = [None] * units
    oh = [None] * units
    waited = [False] * units

    def wait_gh(u):
        if u < 0 or waited[u] or gh[u] is None:
            return
        ent = gh[u]
        if isinstance(ent, tuple):
            cond, cp = ent

            @pl.when(cond)
            def _():
                cp.wait()
        else:
            ent.wait()
        waited[u] = True

    def start_out(u):
        l, r = divmod(u, BEAM * nh)
        j, h = divmod(r, nh)
        cp = pltpu.make_async_copy(ring.at[u % NB],
                                   out_refs[l].at[j, h],
                                   out_sems.at[u % NB])
        cp.start()
        oh[u] = cp

    for u in range(units):
        l, r = divmod(u, BEAM * nh)
        j, h = divmod(r, nh)
        if u >= NB:
            oh[u - NB].wait()
        if j == 0:
            cp = pltpu.make_async_copy(kv_refs[l].at[b[j], h],
                                       ring.at[u % NB],
                                       in_sems.at[u % NB])
            cp.start(priority=u % 2)
            gh[u] = cp
        else:
            is_dup = f[j] < j
            not_dup = jnp.logical_not(is_dup)
            src_slot = (u - (j - f[j]) * nh) % NB
            for back in (nh, 2 * nh):
                if j * nh >= back:
                    wait_gh(u - back)
            cp = pltpu.make_async_copy(kv_refs[l].at[b[j], h],
                                       ring.at[u % NB],
                                       in_sems.at[u % NB])

            @pl.when(not_dup)
            def _():
                cp.start(priority=u % 2)

            @pl.when(is_dup)
            def _():
                ring[pl.ds(u % NB, 1)] = ring[pl.ds(src_slot, 1)]
            gh[u] = (not_dup, cp)
        if u >= LAG:
            wait_gh(u - LAG)
            start_out(u - LAG)

    riota1 = lax.broadcasted_iota(jnp.int32, (BEAM, 1), 0)
    tbi_col = jnp.where(riota1 == 0, t_sel[0],
                        jnp.where(riota1 == 1, t_sel[1], t_sel[2]))
    tbi_ref[...] = tbi_col
    tbp_ref[...] = jnp.where(riota1 == 0, v_sel[0],
                             jnp.where(riota1 == 1, v_sel[1], v_sel[2]))
    riota20 = lax.broadcasted_iota(jnp.int32, (BEAM, HIST), 0)
    sid = jnp.where(riota20 == 0, save_id_ref[pl.ds(b[0], 1), :],
                    jnp.where(riota20 == 1, save_id_ref[pl.ds(b[1], 1), :],
                              save_id_ref[pl.ds(b[2], 1), :]))
    nsi_ref[...] = jnp.concatenate([sid, tbi_col], axis=1)

    col_iota = lax.broadcasted_iota(jnp.int32, (1, VOCAB), 1)
    for j in range(BEAM):
        row = rp_ref[pl.ds(b[j], 1), :]
        row = jnp.where(col_iota == t_sel[j], row * pen_ref[0], row)
        rp_out_ref[pl.ds(j, 1), :] = row

    for u in range(units - LAG, units):
        wait_gh(u)
        start_out(u)
    for u in range(units - NB, units):
        oh[u].wait()


@jax.jit
def _run(kvs, logits, save_id, repeat_penality, previous_prob, penality_value):
    kv_shape = kvs[0].shape
    out_shape = (
        [jax.ShapeDtypeStruct(kv_shape, jnp.float32) for _ in range(N_LAYERS)]
        + [
            jax.ShapeDtypeStruct((BEAM, 1), jnp.int32),
            jax.ShapeDtypeStruct((BEAM, HIST + 1), jnp.int32),
            jax.ShapeDtypeStruct((BEAM, VOCAB), jnp.float32),
            jax.ShapeDtypeStruct((BEAM, 1), jnp.float32),
            jax.ShapeDtypeStruct((1,), jnp.int32),
        ]
    )
    vmem = pl.BlockSpec(memory_space=pltpu.MemorySpace.VMEM)
    smem = pl.BlockSpec(memory_space=pltpu.SMEM)
    hbm = pl.BlockSpec(memory_space=pl.ANY)
    outs = pl.pallas_call(
        _body,
        out_shape=out_shape,
        in_specs=[vmem, vmem, vmem, vmem, smem] + [hbm] * N_LAYERS,
        out_specs=[hbm] * N_LAYERS + [vmem, vmem, vmem, vmem, smem],
        scratch_shapes=[
            pltpu.SMEM((BEAM * TOPK,), jnp.float32),
            pltpu.SMEM((BEAM * TOPK,), jnp.int32),
            pltpu.VMEM((NB, kv_shape[2], kv_shape[3]), jnp.float32),
            pltpu.SemaphoreType.DMA((NB,)),
            pltpu.SemaphoreType.DMA((NB,)),
        ],
    )(logits, save_id, repeat_penality, previous_prob, penality_value, *kvs)
    save_kv = outs[:N_LAYERS]
    tbi, nsi, rp_out, tbp, mli = outs[N_LAYERS:]
    return (*save_kv, tbi, nsi, rp_out, tbp, mli)


def kernel(kv_0, kv_1, kv_2, kv_3, kv_4, kv_5, kv_6, kv_7, kv_8, kv_9,
           kv_10, kv_11, logits, save_id, repeat_penality, previous_prob,
           penality_value, beam_size, topK):
    kvs = (kv_0, kv_1, kv_2, kv_3, kv_4, kv_5, kv_6, kv_7, kv_8, kv_9,
           kv_10, kv_11)
    return _run(kvs, logits, save_id, repeat_penality, previous_prob,
                penality_value)
